# baseline (speedup 1.0000x reference)
import jax
import jax.numpy as jnp
from jax.experimental import pallas as pl
from jax.experimental.pallas import tpu as pltpu


def _mm(a3, w):
    T, B, K = a3.shape
    return jnp.dot(a3.reshape(T * B, K), w,
                   preferred_element_type=jnp.float32).reshape(T, B, w.shape[1])


def _cnn_rnn_body(emb_ref, feat_ref,
                  w1_ref, b1_ref,
                  w2_ref, b2_ref,
                  wih1_ref, whh1_ref, bg1_ref,
                  wihh2_ref, bg2_ref,
                  wfco_ref, wfcf_ref, bfc_ref,
                  out_ref):
    bf16 = jnp.bfloat16
    f32 = jnp.float32
    emb = jnp.transpose(emb_ref[...], (1, 0, 2)).astype(bf16)
    L, B, E = emb.shape
    T = L - 2
    C1 = w1_ref.shape[1]
    H = whh1_ref.shape[0]

    sv = jnp.concatenate([jnp.full((1, 2 * H), 0.5, f32),
                          jnp.ones((1, H), f32),
                          jnp.full((1, H), 0.5, f32)], axis=1)

    w1f = (w1_ref[...] * 0.5).astype(bf16)
    b1f = b1_ref[...] * 0.5
    w2f = (w2_ref[...] * 0.5).astype(bf16)
    b2f = b2_ref[...] * 0.5
    wih1f = (wih1_ref[...] * sv).astype(bf16)
    bg1f = sv * bg1_ref[...]
    whh1f = (whh1_ref[...] * sv).astype(bf16)
    wih2f = (wihh2_ref[0:H, :] * sv).astype(bf16)
    whh2f = (wihh2_ref[H:2 * H, :] * sv).astype(bf16)
    bg2f = bg2_ref[...] * sv
    w1x2f = jnp.concatenate([whh1f, wih2f], axis=1)

    win1 = jnp.concatenate(
        [emb[0:T], emb[1:T + 1], emb[2:T + 2]], axis=-1)
    c1 = (0.5 * jnp.tanh(_mm(win1, w1f) + b1f) + 0.5).astype(bf16)

    zpad = jnp.zeros((1, B, C1), bf16)
    c1p = jnp.concatenate([zpad, c1, zpad], axis=0)
    win2 = jnp.concatenate(
        [c1p[0:T], c1p[1:T + 1], c1p[2:T + 2]], axis=-1)
    c2 = (0.5 * jnp.tanh(_mm(win2, w2f) + b2f) + 0.5).astype(bf16)

    u1x = _mm(c2, wih1f) + bg1f

    def apply_gates(tu, c_prev):
        i = 0.5 * tu[:, 0:H] + 0.5
        f = 0.5 * tu[:, H:2 * H] + 0.5
        g = tu[:, 2 * H:3 * H]
        o = 0.5 * tu[:, 3 * H:4 * H] + 0.5
        c_new = f * c_prev + i * g
        h_new = o * jnp.tanh(c_new)
        return h_new, c_new

    h1 = jnp.zeros((B, H), bf16)
    c1s = jnp.zeros((B, H), f32)
    h2 = jnp.zeros((B, H), bf16)
    c2s = jnp.zeros((B, H), f32)

    hs = []
    for s in range(T + 1):
        h1c = h1
        if s < T:
            g1 = jnp.dot(h1c, w1x2f, preferred_element_type=f32)
            tu1 = jnp.tanh(u1x[s] + g1[:, 0:4 * H])
            h1f, c1s = apply_gates(tu1, c1s)
            h1 = h1f.astype(bf16)
            g2x = g1[:, 4 * H:8 * H]
        else:
            g2x = jnp.dot(h1c, wih2f, preferred_element_type=f32)
        if s >= 1:
            tu2 = jnp.tanh(g2x
                           + jnp.dot(h2, whh2f, preferred_element_type=f32)
                           + bg2f)
            h2f, c2s = apply_gates(tu2, c2s)
            h2 = h2f.astype(bf16)
            hs.append(h2)

    Th = T // 2
    hflat_a = jnp.concatenate(hs[0:Th], axis=-1)
    hflat_b = jnp.concatenate(hs[Th:T], axis=-1)
    out_ref[...] = (jnp.dot(hflat_a, wfco_ref[0:Th * H, :].astype(bf16),
                            preferred_element_type=f32)
                    + jnp.dot(hflat_b, wfco_ref[Th * H:T * H, :].astype(bf16),
                              preferred_element_type=f32)
                    + jnp.dot(feat_ref[...], wfcf_ref[...],
                              preferred_element_type=f32)
                    + bfc_ref[...])


def kernel(emb, feat, w1, b1, w2, b2, wih1, whh1, bg1, wihh2, bg2,
           wfco, wfcf, bfc):
    B = emb.shape[0]
    NL = bfc.shape[1]

    Bp = max(8, ((B + 7) // 8) * 8)
    if Bp != B:
        emb = jnp.pad(emb, ((0, Bp - B), (0, 0), (0, 0)))
        feat = jnp.pad(feat, ((0, Bp - B), (0, 0)))

    inputs = (emb, feat, w1, b1, w2, b2, wih1, whh1, bg1, wihh2, bg2,
              wfco, wfcf, bfc)

    def full_spec(shape):
        nd = len(shape)
        return pl.BlockSpec(shape, lambda i, nd=nd: (0,) * nd)

    out = pl.pallas_call(
        _cnn_rnn_body,
        out_shape=jax.ShapeDtypeStruct((Bp, NL), jnp.float32),
        grid=(1,),
        in_specs=[full_spec(a.shape) for a in inputs],
        out_specs=full_spec((Bp, NL)),
        compiler_params=pltpu.CompilerParams(
            dimension_semantics=("arbitrary",)),
    )(*inputs)
    return out[:B]


# speedup vs baseline: 1.0079x; 1.0079x over previous
import jax
import jax.numpy as jnp
from jax.experimental import pallas as pl
from jax.experimental.pallas import tpu as pltpu


def _mm(a3, w):
    T, B, K = a3.shape
    return jnp.dot(a3.reshape(T * B, K), w,
                   preferred_element_type=jnp.float32).reshape(T, B, w.shape[1])


def _cnn_rnn_body(emb_ref, feat_ref,
                  w1_ref, b1_ref,
                  w2_ref, b2_ref,
                  wih1_ref, whh1_ref, bg1_ref,
                  wihh2_ref, bg2_ref,
                  wfco_ref, wfcf_ref, bfc_ref,
                  out_ref):
    bf16 = jnp.bfloat16
    f32 = jnp.float32
    emb = jnp.transpose(emb_ref[...], (1, 0, 2)).astype(bf16)
    L, B, E = emb.shape
    T = L - 2
    C1 = w1_ref.shape[1]
    H = whh1_ref.shape[0]

    sv = jnp.concatenate([jnp.full((1, 2 * H), 0.5, f32),
                          jnp.ones((1, H), f32),
                          jnp.full((1, H), 0.5, f32)], axis=1)

    w1f = (w1_ref[...] * 0.5).astype(bf16)
    b1f = b1_ref[...] * 0.5
    w2f = (w2_ref[...] * 0.5).astype(bf16)
    b2f = b2_ref[...] * 0.5
    wih1f = (wih1_ref[...] * sv).astype(bf16)
    bg1f = sv * bg1_ref[...]
    whh1f = (whh1_ref[...] * sv).astype(bf16)
    wih2f = (wihh2_ref[0:H, :] * sv).astype(bf16)
    whh2f = (wihh2_ref[H:2 * H, :] * sv).astype(bf16)
    bg2f = bg2_ref[...] * sv
    w1x2f = jnp.concatenate([whh1f, wih2f], axis=1)

    win1 = jnp.concatenate(
        [emb[0:T], emb[1:T + 1], emb[2:T + 2]], axis=-1)
    c1 = (0.5 * jnp.tanh(_mm(win1, w1f) + b1f) + 0.5).astype(bf16)

    zpad = jnp.zeros((1, B, C1), bf16)
    c1p = jnp.concatenate([zpad, c1, zpad], axis=0)
    win2 = jnp.concatenate(
        [c1p[0:T], c1p[1:T + 1], c1p[2:T + 2]], axis=-1)
    c2 = (0.5 * jnp.tanh(_mm(win2, w2f) + b2f) + 0.5).astype(bf16)

    u1x = _mm(c2, wih1f) + bg1f

    def apply_gates(tu, c_prev):
        i = 0.5 * tu[:, 0:H] + 0.5
        f = 0.5 * tu[:, H:2 * H] + 0.5
        g = tu[:, 2 * H:3 * H]
        o = 0.5 * tu[:, 3 * H:4 * H] + 0.5
        c_new = f * c_prev + i * g
        h_new = o * jnp.tanh(c_new)
        return h_new, c_new

    h1 = jnp.zeros((B, H), bf16)
    c1s = jnp.zeros((B, H), f32)
    h2 = jnp.zeros((B, H), bf16)
    c2s = jnp.zeros((B, H), f32)

    hs = []
    for s in range(T + 1):
        h1c = h1
        if s < T:
            g1 = jnp.dot(h1c, w1x2f, preferred_element_type=f32)
            tu1 = jnp.tanh(u1x[s] + g1[:, 0:4 * H])
            h1f, c1s = apply_gates(tu1, c1s)
            h1 = h1f.astype(bf16)
            g2x = g1[:, 4 * H:8 * H]
        else:
            g2x = jnp.dot(h1c, wih2f, preferred_element_type=f32)
        if s >= 1:
            tu2 = jnp.tanh(g2x
                           + jnp.dot(h2, whh2f, preferred_element_type=f32)
                           + bg2f)
            h2f, c2s = apply_gates(tu2, c2s)
            h2 = h2f.astype(bf16)
            hs.append(h2)

    hflat = jnp.concatenate(hs, axis=-1)
    out_ref[...] = (jnp.dot(hflat, wfco_ref[...].astype(bf16),
                            preferred_element_type=f32)
                    + jnp.dot(feat_ref[...], wfcf_ref[...],
                              preferred_element_type=f32)
                    + bfc_ref[...])


def kernel(emb, feat, w1, b1, w2, b2, wih1, whh1, bg1, wihh2, bg2,
           wfco, wfcf, bfc):
    B = emb.shape[0]
    NL = bfc.shape[1]

    Bp = max(8, ((B + 7) // 8) * 8)
    if Bp != B:
        emb = jnp.pad(emb, ((0, Bp - B), (0, 0), (0, 0)))
        feat = jnp.pad(feat, ((0, Bp - B), (0, 0)))

    inputs = (emb, feat, w1, b1, w2, b2, wih1, whh1, bg1, wihh2, bg2,
              wfco, wfcf, bfc)

    def full_spec(shape):
        nd = len(shape)
        return pl.BlockSpec(shape, lambda i, nd=nd: (0,) * nd)

    out = pl.pallas_call(
        _cnn_rnn_body,
        out_shape=jax.ShapeDtypeStruct((Bp, NL), jnp.float32),
        grid=(1,),
        in_specs=[full_spec(a.shape) for a in inputs],
        out_specs=full_spec((Bp, NL)),
        compiler_params=pltpu.CompilerParams(
            dimension_semantics=("arbitrary",)),
    )(*inputs)
    return out[:B]
